# baseline (speedup 1.0000x reference)
import functools

import jax
import jax.numpy as jnp
from jax import lax
from jax.experimental import pallas as pl
from jax.experimental.pallas import tpu as pltpu
from jax.experimental.pallas import tpu_sc as plsc

N = 100000
D = 64
MAX_DEPTH = 20
BLK = 128
NBLK = (N + BLK - 1) // BLK
NBUF = 4
PF = 3

_info = plsc.get_sparse_core_info()
NC, NS = _info.num_cores, _info.num_subcores
NW = NC * NS

BASE_BLKS = NBLK // NW
EXTRA = NBLK - BASE_BLKS * NW
MAX_BLKS = BASE_BLKS + 1
WIN = MAX_BLKS * BLK

_mesh = plsc.VectorSubcoreMesh(core_axis_name="c", subcore_axis_name="s")


@functools.partial(
    pl.kernel,
    mesh=_mesh,
    out_type=jax.ShapeDtypeStruct((N, D), jnp.float32),
    compiler_params=pltpu.CompilerParams(use_tc_tiling_on_sc=False),
    scratch_types=[
        pltpu.VMEM((WIN,), jnp.int32),
        pltpu.VMEM((WIN,), jnp.int32),
        pltpu.VMEM((WIN,), jnp.int32),
        pltpu.VMEM((WIN,), jnp.int32),
    ]
    + [pltpu.VMEM((BLK, D), jnp.float32)] * (3 * NBUF)
    + [pltpu.SemaphoreType.DMA] * (1 + 2 * NBUF),
)
def _encode(x0_hbm, x1_hbm, dep_hbm, ttab, atab, dtab, out_hbm,
            idx0_v_unused, idx0_v, idx1_v, dep_v,
            t0, t1, t2, t3, a0, a1, a2, a3, d0, d1, d2, d3,
            ssem, g0sem, g1sem, g2sem, g3sem, w0sem, w1sem, w2sem, w3sem):
    t_bufs = (t0, t1, t2, t3)
    a_bufs = (a0, a1, a2, a3)
    d_bufs = (d0, d1, d2, d3)
    gsems = (g0sem, g1sem, g2sem, g3sem)
    wsems = (w0sem, w1sem, w2sem, w3sem)

    wid = lax.axis_index("s") * NC + lax.axis_index("c")
    first_blk = wid * BASE_BLKS + lax.min(wid, EXTRA)
    n_blk = BASE_BLKS + jnp.where(wid < EXTRA, 1, 0)
    start = lax.min(first_blk * BLK, N - WIN)

    c0 = pltpu.async_copy(x0_hbm.at[pl.ds(start, WIN)], idx0_v, ssem)
    c1 = pltpu.async_copy(x1_hbm.at[pl.ds(start, WIN)], idx1_v, ssem)
    c2 = pltpu.async_copy(dep_hbm.at[pl.ds(start, WIN)], dep_v, ssem)
    c0.wait()
    c1.wait()
    c2.wait()

    iota = lax.iota(jnp.int32, 16)
    zeros = jnp.zeros((16,), jnp.int32)
    ones = jnp.ones((16,), jnp.int32)

    def prep(i, carry):
        s = pl.ds(i * 16, 16)
        rows = i * 16 + iota
        dep_v[s] = jnp.minimum(dep_v[s], MAX_DEPTH)
        return carry

    lax.fori_loop(0, WIN // 16, prep, 0)

    def lbase(k):
        return lax.min((first_blk + k) * BLK, N - BLK) - start

    def issue(k, b):
        lb = lbase(k)
        pltpu.async_copy(ttab.at[idx0_v.at[pl.ds(lb, BLK)]], t_bufs[b], gsems[b])
        pltpu.async_copy(atab.at[idx1_v.at[pl.ds(lb, BLK)]], a_bufs[b], gsems[b])
        pltpu.async_copy(dtab.at[dep_v.at[pl.ds(lb, BLK)]], d_bufs[b], gsems[b])

    for k in range(PF):
        issue(k, k)

    for k in range(MAX_BLKS):
        b = k % NBUF
        pk = k + PF
        pb = pk % NBUF
        if pk < MAX_BLKS:
            def prefetch():
                if pk >= NBUF:
                    pltpu.make_async_copy(
                        a_bufs[pb], out_hbm.at[pl.ds(0, BLK)], wsems[pb]).wait()
                issue(pk, pb)

            if pk < BASE_BLKS:
                prefetch()
            else:
                pl.when(pk < n_blk)(prefetch)

        def compute():
            pltpu.make_async_copy(
                ttab.at[idx0_v.at[pl.ds(0, BLK)]], t_bufs[b], gsems[b]).wait()
            pltpu.make_async_copy(
                atab.at[idx1_v.at[pl.ds(0, BLK)]], a_bufs[b], gsems[b]).wait()
            pltpu.make_async_copy(
                dtab.at[dep_v.at[pl.ds(0, BLK)]], d_bufs[b], gsems[b]).wait()

            def row(r, carry):
                for c in range(D // 16):
                    s = pl.ds(c * 16, 16)
                    a_bufs[b][r, s] = a_bufs[b][r, s] + t_bufs[b][r, s] + d_bufs[b][r, s]
                return carry

            lax.fori_loop(0, BLK, row, 0)
            gb = lax.min((first_blk + k) * BLK, N - BLK)
            pltpu.async_copy(a_bufs[b], out_hbm.at[pl.ds(gb, BLK)], wsems[b])

        if k < BASE_BLKS:
            compute()
        else:
            pl.when(k < n_blk)(compute)

    for kk in range(BASE_BLKS - NBUF, MAX_BLKS):
        b = kk % NBUF

        def drain(b=b):
            pltpu.make_async_copy(
                a_bufs[b], out_hbm.at[pl.ds(0, BLK)], wsems[b]).wait()

        pl.when(jnp.logical_and(kk >= n_blk - NBUF, kk < n_blk))(drain)


def kernel(x, depth, type_table, attribute_table, depth_table):
    return _encode(x[:, 0], x[:, 1], depth, type_table, attribute_table, depth_table)


# speedup vs baseline: 1.1663x; 1.1663x over previous
import functools

import jax
import jax.numpy as jnp
from jax import lax
from jax.experimental import pallas as pl
from jax.experimental.pallas import tpu as pltpu
from jax.experimental.pallas import tpu_sc as plsc

N = 100000
D = 64
MAX_DEPTH = 20
BLK = 80
PBLK = BLK // 2
NBLK = N // BLK
NBUF = 3
PF = 2

TROWS = 500
AROWS = 500000
DROWS = 11

_info = plsc.get_sparse_core_info()
NC, NS = _info.num_cores, _info.num_subcores
NW = NC * NS

BASE_BLKS = NBLK // NW
EXTRA = NBLK - BASE_BLKS * NW
MAX_BLKS = BASE_BLKS + 1
WIN = MAX_BLKS * BLK

_mesh = plsc.VectorSubcoreMesh(core_axis_name="c", subcore_axis_name="s")


@functools.partial(
    pl.kernel,
    mesh=_mesh,
    out_type=jax.ShapeDtypeStruct((N // 2, 128), jnp.float32),
    compiler_params=pltpu.CompilerParams(use_tc_tiling_on_sc=True, needs_layout_passes=False),
    scratch_types=[
        pltpu.VMEM((WIN,), jnp.int32),
        pltpu.VMEM((WIN,), jnp.int32),
        pltpu.VMEM((WIN,), jnp.int32),
        pltpu.VMEM((WIN,), jnp.int32),
        pltpu.VMEM((TROWS * 128,), jnp.float32),
        pltpu.VMEM((DROWS * 128,), jnp.float32),
    ]
    + [pltpu.VMEM((BLK, 128), jnp.float32)] * NBUF
    + [pltpu.VMEM((PBLK, 128), jnp.float32)] * NBUF
    + [pltpu.SemaphoreType.DMA] * (1 + 2 * NBUF),
)
def _encode(x0_hbm, x1_hbm, dep_hbm, ttab, atab, dtab, out_hbm,
            xw0, xw1, dw, pw, tv, dv,
            a0, a1, a2, o0, o1, o2,
            ssem, g0sem, g1sem, g2sem, w0sem, w1sem, w2sem):
    a_bufs = (a0, a1, a2)
    o_bufs = (o0, o1, o2)
    gsems = (g0sem, g1sem, g2sem)
    wsems = (w0sem, w1sem, w2sem)

    wid = lax.axis_index("s") * NC + lax.axis_index("c")
    first_blk = wid * BASE_BLKS + lax.min(wid, EXTRA)
    n_blk = BASE_BLKS + jnp.where(wid < EXTRA, 1, 0)
    start = lax.min(first_blk * BLK, N - WIN)

    c0 = pltpu.async_copy(x0_hbm.at[pl.ds(start, WIN)], xw0, ssem)
    c1 = pltpu.async_copy(x1_hbm.at[pl.ds(start, WIN)], xw1, ssem)
    c2 = pltpu.async_copy(dep_hbm.at[pl.ds(start, WIN)], dw, ssem)
    c3 = pltpu.async_copy(ttab, tv, ssem)
    c4 = pltpu.async_copy(dtab, dv, ssem)
    c0.wait()
    c1.wait()
    c2.wait()
    c3.wait()
    c4.wait()

    def prep(i, carry):
        s = pl.ds(i * 16, 16)
        dw[s] = jnp.minimum(dw[s], MAX_DEPTH)
        pw[s] = jnp.right_shift(xw1[s], 1)
        return carry

    lax.fori_loop(0, WIN // 16, prep, 0)

    def lbase(k):
        return lax.min((first_blk + k) * BLK, N - BLK) - start

    def issue(k, b):
        lb = lbase(k)
        pltpu.async_copy(atab.at[pw.at[pl.ds(lb, BLK)]], a_bufs[b], gsems[b])

    iota = lax.iota(jnp.int32, 16)

    def compute(k, b):
        pltpu.make_async_copy(
            atab.at[pw.at[pl.ds(0, BLK)]], a_bufs[b], gsems[b]).wait()
        lb = lbase(k)
        ab = a_bufs[b]
        ob = o_bufs[b]
        for g in range(BLK // 16):
            s = pl.ds(lb + g * 16, 16)
            x0c = xw0[s]
            tfl = jnp.left_shift(x0c, 6)
            aoff = jnp.left_shift(jnp.bitwise_and(xw1[s], 1), 6)
            dc = dw[s]
            dfl = jnp.left_shift(dc, 6)
            rl = g * 16 + iota
            orow = jnp.right_shift(rl, 1)
            ocol = jnp.left_shift(jnp.bitwise_and(rl, 1), 6)

            def col4(c4, carry):
                for u in range(4):
                    c = c4 * 4 + u
                    av = plsc.load_gather(ab, [rl, aoff + c])
                    tvv = plsc.load_gather(tv, [tfl + c])
                    dvv = plsc.load_gather(dv, [dfl + c])
                    plsc.store_scatter(ob, [orow, ocol + c], av + tvv + dvv)
                return carry

            lax.fori_loop(0, 16, col4, 0)
        pb = lax.min((first_blk + k) * PBLK, N // 2 - PBLK)
        pltpu.async_copy(ob, out_hbm.at[pl.ds(pb, PBLK)], wsems[b])

    for k in range(PF):
        issue(k, k)

    for i in range(NBUF):
        @pl.when(i + PF < n_blk)
        def _(i=i):
            issue(i + PF, (i + PF) % NBUF)

        @pl.when(i < n_blk)
        def _(i=i):
            compute(i, i)

    n_grp = (MAX_BLKS + NBUF - 1) // NBUF

    def group(gidx, carry):
        for i in range(NBUF):
            k = gidx * NBUF + i

            @pl.when(k + PF < n_blk)
            def _(i=i, k=k):
                issue(k + PF, (i + PF) % NBUF)

            @pl.when(k < n_blk)
            def _(i=i, k=k):
                pltpu.make_async_copy(
                    o_bufs[i], out_hbm.at[pl.ds(0, PBLK)], wsems[i]).wait()
                compute(k, i)

        return carry

    lax.fori_loop(1, n_grp, group, 0)

    for kk in range(BASE_BLKS - NBUF, MAX_BLKS):
        b = kk % NBUF

        def drain(b=b):
            pltpu.make_async_copy(
                o_bufs[b], out_hbm.at[pl.ds(0, PBLK)], wsems[b]).wait()

        pl.when(jnp.logical_and(kk >= n_blk - NBUF, kk < n_blk))(drain)


def kernel(x, depth, type_table, attribute_table, depth_table):
    x0 = x[:, 0]
    x1 = x[:, 1]
    t2 = type_table.reshape(TROWS * 128)
    a2 = attribute_table.reshape(AROWS, 128)
    d2 = jnp.pad(depth_table, ((0, 1), (0, 0))).reshape(DROWS * 128)
    out = _encode(x0, x1, depth, t2, a2, d2)
    return out.reshape(N, D)


# speedup vs baseline: 1.2885x; 1.1048x over previous
import functools

import jax
import jax.numpy as jnp
from jax import lax
from jax.experimental import pallas as pl
from jax.experimental.pallas import tpu as pltpu
from jax.experimental.pallas import tpu_sc as plsc

N = 100000
D = 64
MAX_DEPTH = 20
BLK = 64
NBLK = (N + BLK - 1) // BLK
NBUF = 3
PF = 2

TROWS = 1000
DROWS = 21

_info = plsc.get_sparse_core_info()
NC, NS = _info.num_cores, _info.num_subcores
NW = NC * NS

BASE_BLKS = NBLK // NW
EXTRA = NBLK - BASE_BLKS * NW
MAX_BLKS = BASE_BLKS + 1
WIN = MAX_BLKS * BLK

_mesh = plsc.VectorSubcoreMesh(core_axis_name="c", subcore_axis_name="s")


@functools.partial(
    pl.kernel,
    mesh=_mesh,
    out_type=jax.ShapeDtypeStruct((N, 128), jnp.float32),
    compiler_params=pltpu.CompilerParams(
        use_tc_tiling_on_sc=True, needs_layout_passes=False),
    scratch_types=[
        pltpu.VMEM((WIN,), jnp.int32),
        pltpu.VMEM((WIN,), jnp.int32),
        pltpu.VMEM((WIN,), jnp.int32),
        pltpu.VMEM((TROWS * 64,), jnp.float32),
        pltpu.VMEM((DROWS * 64,), jnp.float32),
    ]
    + [pltpu.VMEM((BLK, 128), jnp.float32)] * NBUF
    + [pltpu.VMEM((BLK, 128), jnp.float32)] * NBUF
    + [pltpu.SemaphoreType.DMA] * (1 + 2 * NBUF),
)
def _encode(x0_hbm, x1_hbm, dep_hbm, ttab, atab, dtab, out_hbm,
            xw0, xw1, dw, tv, dv,
            a0, a1, a2, o0, o1, o2,
            ssem, g0sem, g1sem, g2sem, w0sem, w1sem, w2sem):
    a_bufs = (a0, a1, a2)
    o_bufs = (o0, o1, o2)
    gsems = (g0sem, g1sem, g2sem)
    wsems = (w0sem, w1sem, w2sem)

    wid = lax.axis_index("s") * NC + lax.axis_index("c")
    first_blk = wid * BASE_BLKS + lax.min(wid, EXTRA)
    n_blk = BASE_BLKS + jnp.where(wid < EXTRA, 1, 0)
    start = lax.min(first_blk * BLK, N - WIN)

    c0 = pltpu.async_copy(x0_hbm.at[pl.ds(start, WIN)], xw0, ssem)
    c1 = pltpu.async_copy(x1_hbm.at[pl.ds(start, WIN)], xw1, ssem)
    c2 = pltpu.async_copy(dep_hbm.at[pl.ds(start, WIN)], dw, ssem)
    c3 = pltpu.async_copy(ttab, tv, ssem)
    c4 = pltpu.async_copy(dtab, dv, ssem)
    c0.wait()
    c1.wait()
    c2.wait()
    c3.wait()
    c4.wait()

    def prep(i, carry):
        s = pl.ds(i * 16, 16)
        dw[s] = jnp.minimum(dw[s], MAX_DEPTH)
        return carry

    lax.fori_loop(0, WIN // 16, prep, 0)

    def lbase(k):
        return lax.min((first_blk + k) * BLK, N - BLK) - start

    def issue(k, b):
        lb = lbase(k)
        pltpu.async_copy(atab.at[xw1.at[pl.ds(lb, BLK)]], a_bufs[b], gsems[b])

    iota = lax.iota(jnp.int32, 16)

    def compute(k, b):
        pltpu.make_async_copy(
            atab.at[xw1.at[pl.ds(0, BLK)]], a_bufs[b], gsems[b]).wait()
        lb = lbase(k)
        ab = a_bufs[b]
        ob = o_bufs[b]
        for g in range(BLK // 16):
            s = pl.ds(lb + g * 16, 16)
            tfl = jnp.left_shift(xw0[s], 6)
            dfl = jnp.left_shift(dw[s], 6)
            rl = g * 16 + iota

            def col4(c4, carry):
                for u in range(4):
                    c = c4 * 4 + u
                    cv = jnp.full((16,), c, jnp.int32)
                    av = plsc.load_gather(ab, [rl, cv])
                    tvv = plsc.load_gather(tv, [tfl + c])
                    dvv = plsc.load_gather(dv, [dfl + c])
                    plsc.store_scatter(ob, [rl, cv], av + tvv + dvv)
                return carry

            lax.fori_loop(0, 16, col4, 0)
        gb = lax.min((first_blk + k) * BLK, N - BLK)
        pltpu.async_copy(ob, out_hbm.at[pl.ds(gb, BLK)], wsems[b])

    for k in range(PF):
        issue(k, k)

    for i in range(NBUF):
        @pl.when(i + PF < n_blk)
        def _(i=i):
            issue(i + PF, (i + PF) % NBUF)

        @pl.when(i < n_blk)
        def _(i=i):
            compute(i, i)

    n_grp = (MAX_BLKS + NBUF - 1) // NBUF

    def group(gidx, carry):
        for i in range(NBUF):
            k = gidx * NBUF + i

            @pl.when(k + PF < n_blk)
            def _(i=i, k=k):
                issue(k + PF, (i + PF) % NBUF)

            @pl.when(k < n_blk)
            def _(i=i, k=k):
                pltpu.make_async_copy(
                    o_bufs[i], out_hbm.at[pl.ds(0, BLK)], wsems[i]).wait()
                compute(k, i)

        return carry

    lax.fori_loop(1, n_grp, group, 0)

    for kk in range(BASE_BLKS - NBUF, MAX_BLKS):
        b = kk % NBUF

        def drain(b=b):
            pltpu.make_async_copy(
                o_bufs[b], out_hbm.at[pl.ds(0, BLK)], wsems[b]).wait()

        pl.when(jnp.logical_and(kk >= n_blk - NBUF, kk < n_blk))(drain)


def kernel(x, depth, type_table, attribute_table, depth_table):
    x0 = x[:, 0]
    x1 = x[:, 1]
    t2 = type_table.reshape(TROWS * 64)
    a2 = jnp.pad(attribute_table, ((0, 0), (0, 64)))
    d2 = depth_table.reshape(DROWS * 64)
    out = _encode(x0, x1, depth, t2, a2, d2)
    return out[:, :D]
